# baseline (speedup 1.0000x reference)
import functools

import jax
import jax.numpy as jnp
from jax import lax
from jax.experimental import pallas as pl
from jax.experimental.pallas import tpu as pltpu
from jax.experimental.pallas import tpu_sc as plsc

N = 10000
N_PAD = 10240
E = 160000
NFEAT = 500
W_COL = 16

NC = 2
NS = 16
NW = NC * NS
CHUNK = 128
EPW = E // NW
CPW = (EPW + CHUNK - 1) // CHUNK
EBUF = CPW * CHUNK
RPS = N_PAD // NS
NBUF = 8


def _splat(vec, lane):
    return lax.gather(
        vec,
        jnp.full((W_COL, 1), lane, jnp.int32),
        lax.GatherDimensionNumbers(
            offset_dims=(),
            collapsed_slice_dims=(0,),
            start_index_map=(0,),
        ),
        slice_sizes=(1,),
        mode=lax.GatherScatterMode.PROMISE_IN_BOUNDS,
    )


def _spmm_sc(edge_index, adj_vals, sup, w=None):
    fused = w is not None
    mesh = plsc.VectorSubcoreMesh(
        core_axis_name="c", subcore_axis_name="s", num_cores=NC, num_subcores=NS
    )

    scratch = [
        pltpu.VMEM((EBUF,), jnp.int32),
        pltpu.VMEM((EBUF,), jnp.int32),
        pltpu.VMEM((EBUF,), jnp.float32),
        pltpu.VMEM((NBUF, CHUNK, W_COL), jnp.float32),
        pltpu.VMEM((NBUF, CHUNK, W_COL), jnp.float32),
        pltpu.VMEM((RPS, W_COL), jnp.float32),
        pltpu.VMEM_SHARED((N_PAD, W_COL), jnp.float32),
        pltpu.VMEM_SHARED((N_PAD, W_COL), jnp.float32),
        [pltpu.SemaphoreType.DMA] * NBUF,
        [pltpu.SemaphoreType.DMA] * NBUF,
        [pltpu.SemaphoreType.DMA] * 6,
    ]
    if fused:
        scratch += [
            pltpu.VMEM((RPS, W_COL), jnp.float32),
            pltpu.VMEM((RPS, W_COL), jnp.float32),
            pltpu.VMEM((W_COL, W_COL), jnp.float32),
        ]

    @functools.partial(
        pl.kernel,
        out_type=jax.ShapeDtypeStruct((NC, N_PAD, W_COL), jnp.float32),
        mesh=mesh,
        compiler_params=pltpu.CompilerParams(use_tc_tiling_on_sc=False),
        scratch_types=scratch,
    )
    def spmm(ei_hbm, adj_hbm, sup_hbm, *rest):
        if fused:
            (w_hbm, out_hbm,
             src_v, dst_v, adj_v, gbuf, sbuf, drain_v, acc_sh, sup_sh,
             gsem, ssem, stsem, pb0, pb1, wv) = rest
        else:
            (out_hbm,
             src_v, dst_v, adj_v, gbuf, sbuf, drain_v, acc_sh, sup_sh,
             gsem, ssem, stsem) = rest
        c = lax.axis_index("c")
        s = lax.axis_index("s")
        wid = s * NC + c

        tail0 = (EPW // W_COL) * W_COL
        for t in range(tail0, EBUF, W_COL):
            src_v[pl.ds(t, W_COL)] = jnp.zeros((W_COL,), jnp.int32)
            dst_v[pl.ds(t, W_COL)] = jnp.zeros((W_COL,), jnp.int32)
            adj_v[pl.ds(t, W_COL)] = jnp.zeros((W_COL,), jnp.float32)
        st1 = pltpu.make_async_copy(
            ei_hbm.at[1, pl.ds(wid * EPW, EPW)], src_v.at[pl.ds(0, EPW)],
            stsem[0])
        st2 = pltpu.make_async_copy(
            ei_hbm.at[0, pl.ds(wid * EPW, EPW)], dst_v.at[pl.ds(0, EPW)],
            stsem[1])
        st3 = pltpu.make_async_copy(
            adj_hbm.at[pl.ds(wid * EPW, EPW)], adj_v.at[pl.ds(0, EPW)],
            stsem[2])
        st1.start()
        st2.start()
        st3.start()
        if fused:
            st4 = pltpu.make_async_copy(
                sup_hbm.at[0, pl.ds(s * RPS, RPS)], pb0, stsem[3])
            st5 = pltpu.make_async_copy(
                sup_hbm.at[1, pl.ds(s * RPS, RPS)], pb1, stsem[4])
            st6 = pltpu.make_async_copy(w_hbm, wv, stsem[5])
        else:
            st4 = pltpu.make_async_copy(
                sup_hbm.at[pl.ds(s * RPS, RPS)],
                sup_sh.at[pl.ds(s * RPS, RPS)], stsem[3])
        st4.start()
        if fused:
            st5.start()
            st6.start()

        def zero_body(i, carry):
            drain_v[i, :] = jnp.zeros((W_COL,), jnp.float32)
            return carry
        lax.fori_loop(0, RPS, zero_body, 0)
        pltpu.sync_copy(drain_v, acc_sh.at[pl.ds(s * RPS, RPS)])
        st1.wait()
        st2.wait()
        st3.wait()
        st4.wait()
        if fused:
            st5.wait()
            st6.wait()
            wrows = [wv[k, :] for k in range(W_COL)]

            def mm_body(r, carry):
                h = jnp.maximum(pb0[r, :] + pb1[r, :], 0.0)
                acc = _splat(h, 0) * wrows[0]
                for k in range(1, W_COL):
                    acc = acc + _splat(h, k) * wrows[k]
                drain_v[r, :] = acc
                return carry
            lax.fori_loop(0, RPS, mm_body, 0)
            pltpu.sync_copy(drain_v, sup_sh.at[pl.ds(s * RPS, RPS)])

        def start_gather(j, b):
            pltpu.make_async_copy(
                sup_sh.at[src_v.at[pl.ds(j * CHUNK, CHUNK)]],
                gbuf.at[b], gsem[b]).start()

        def wait_gather(j, b):
            pltpu.make_async_copy(
                sup_sh.at[src_v.at[pl.ds(j * CHUNK, CHUNK)]],
                gbuf.at[b], gsem[b]).wait()

        def start_scatter(j, b):
            pltpu.make_async_copy(
                sbuf.at[b], acc_sh.at[dst_v.at[pl.ds(j * CHUNK, CHUNK)]],
                ssem[b]).start(add=True)

        def wait_scatter(j, b):
            pltpu.make_async_copy(
                sbuf.at[b], acc_sh.at[dst_v.at[pl.ds(j * CHUNK, CHUNK)]],
                ssem[b]).wait()

        def scale(j, b):
            for g in range(CHUNK // W_COL):
                a = adj_v[pl.ds(j * CHUNK + g * W_COL, W_COL)]
                for l in range(W_COL):
                    e = g * W_COL + l
                    sbuf[b, e, :] = gbuf[b, e, :] * _splat(a, l)

        plsc.subcore_barrier()
        for b in range(NBUF):
            start_gather(b, b)

        for j in range(NBUF):
            b = j % NBUF
            wait_gather(j, b)
            scale(j, b)
            start_scatter(j, b)
            start_gather(j + NBUF, b)

        def steady(g, carry):
            for b in range(NBUF):
                j = NBUF + g * NBUF + b
                wait_gather(j, b)
                wait_scatter(j, b)
                scale(j, b)
                start_scatter(j, b)
                start_gather(j + NBUF, b)
            return carry
        lax.fori_loop(0, CPW // NBUF - 2, steady, 0)

        for j in range(CPW - NBUF, CPW):
            b = j % NBUF
            wait_gather(j, b)
            wait_scatter(j, b)
            scale(j, b)
            start_scatter(j, b)

        for j in range(CPW - NBUF, CPW):
            wait_scatter(j, j % NBUF)

        plsc.subcore_barrier()
        pltpu.sync_copy(acc_sh.at[pl.ds(s * RPS, RPS)], drain_v)
        pltpu.sync_copy(drain_v, out_hbm.at[c, pl.ds(s * RPS, RPS)])

    if fused:
        return spmm(edge_index, adj_vals, sup, w)
    return spmm(edge_index, adj_vals, sup)


def _mm_x_w1(x, w1):
    bm = 1000

    def body(x_ref, w_ref, o_ref):
        o_ref[:] = jnp.dot(x_ref[:], w_ref[:],
                           preferred_element_type=jnp.float32)

    return pl.pallas_call(
        body,
        grid=(N // bm,),
        in_specs=[
            pl.BlockSpec((bm, NFEAT), lambda i: (i, 0)),
            pl.BlockSpec((NFEAT, W_COL), lambda i: (0, 0)),
        ],
        out_specs=pl.BlockSpec((bm, W_COL), lambda i: (i, 0)),
        out_shape=jax.ShapeDtypeStruct((N_PAD, W_COL), jnp.float32),
    )(x, w1)


def _sum_log_softmax(p, nclass):
    bm = 1000

    def body(p_ref, o_ref):
        z = p_ref[0] + p_ref[1]
        col = lax.broadcasted_iota(jnp.int32, (bm, W_COL), 1)
        valid = col < nclass
        zm = jnp.where(valid, z, -jnp.inf)
        m = jnp.max(zm, axis=1, keepdims=True)
        ez = jnp.where(valid, jnp.exp(z - m), 0.0)
        ssum = jnp.sum(ez, axis=1, keepdims=True)
        o_ref[:] = (z - m - jnp.log(ssum))[:, :o_ref.shape[1]]

    return pl.pallas_call(
        body,
        grid=(N // bm,),
        in_specs=[pl.BlockSpec((NC, bm, W_COL), lambda i: (0, i, 0))],
        out_specs=pl.BlockSpec((bm, nclass), lambda i: (i, 0)),
        out_shape=jax.ShapeDtypeStruct((N, nclass), jnp.float32),
    )(p)


def kernel(x, edge_index, adj_vals, W1, W2, W3):
    nclass = W3.shape[1]
    w3p = jnp.pad(W3, ((0, 0), (0, W_COL - nclass)))

    sup = _mm_x_w1(x, W1)
    p = _spmm_sc(edge_index, adj_vals, sup)
    p = _spmm_sc(edge_index, adj_vals, p, W2)
    p = _spmm_sc(edge_index, adj_vals, p, w3p)
    return _sum_log_softmax(p, nclass)


# speedup vs baseline: 1.0616x; 1.0616x over previous
import functools

import jax
import jax.numpy as jnp
from jax import lax
from jax.experimental import pallas as pl
from jax.experimental.pallas import tpu as pltpu
from jax.experimental.pallas import tpu_sc as plsc

N = 10000
N_PAD = 10240
E = 160000
NFEAT = 500
W_COL = 16

NC = 2
NS = 16
NW = NC * NS
CHUNK = 128
EPW = E // NW
CPW = (EPW + CHUNK - 1) // CHUNK
EBUF = CPW * CHUNK
RPS = N_PAD // NS
NBUF = 4


def _splat(vec, lane):
    return lax.gather(
        vec,
        jnp.full((W_COL, 1), lane, jnp.int32),
        lax.GatherDimensionNumbers(
            offset_dims=(),
            collapsed_slice_dims=(0,),
            start_index_map=(0,),
        ),
        slice_sizes=(1,),
        mode=lax.GatherScatterMode.PROMISE_IN_BOUNDS,
    )


def _spmm_sc(edge_index, adj_vals, sup, w=None):
    fused = w is not None
    mesh = plsc.VectorSubcoreMesh(
        core_axis_name="c", subcore_axis_name="s", num_cores=NC, num_subcores=NS
    )

    scratch = [
        pltpu.VMEM((EBUF,), jnp.int32),
        pltpu.VMEM((EBUF,), jnp.int32),
        pltpu.VMEM((EBUF,), jnp.float32),
        pltpu.VMEM((NBUF, CHUNK, W_COL), jnp.float32),
        pltpu.VMEM((NBUF, CHUNK, W_COL), jnp.float32),
        pltpu.VMEM((RPS, W_COL), jnp.float32),
        pltpu.VMEM_SHARED((N_PAD, W_COL), jnp.float32),
        pltpu.VMEM_SHARED((N_PAD, W_COL), jnp.float32),
        [pltpu.SemaphoreType.DMA] * NBUF,
        [pltpu.SemaphoreType.DMA] * NBUF,
        [pltpu.SemaphoreType.DMA] * 6,
    ]
    if fused:
        scratch += [
            pltpu.VMEM((RPS, W_COL), jnp.float32),
            pltpu.VMEM((RPS, W_COL), jnp.float32),
            pltpu.VMEM((W_COL, W_COL), jnp.float32),
        ]

    @functools.partial(
        pl.kernel,
        out_type=jax.ShapeDtypeStruct((NC, N_PAD, W_COL), jnp.float32),
        mesh=mesh,
        compiler_params=pltpu.CompilerParams(use_tc_tiling_on_sc=False),
        scratch_types=scratch,
    )
    def spmm(ei_hbm, adj_hbm, sup_hbm, *rest):
        if fused:
            (w_hbm, out_hbm,
             src_v, dst_v, adj_v, gbuf, sbuf, drain_v, acc_sh, sup_sh,
             gsem, ssem, stsem, pb0, pb1, wv) = rest
        else:
            (out_hbm,
             src_v, dst_v, adj_v, gbuf, sbuf, drain_v, acc_sh, sup_sh,
             gsem, ssem, stsem) = rest
        c = lax.axis_index("c")
        s = lax.axis_index("s")
        wid = s * NC + c

        tail0 = (EPW // W_COL) * W_COL
        for t in range(tail0, EBUF, W_COL):
            src_v[pl.ds(t, W_COL)] = jnp.zeros((W_COL,), jnp.int32)
            dst_v[pl.ds(t, W_COL)] = jnp.zeros((W_COL,), jnp.int32)
            adj_v[pl.ds(t, W_COL)] = jnp.zeros((W_COL,), jnp.float32)
        st1 = pltpu.make_async_copy(
            ei_hbm.at[1, pl.ds(wid * EPW, EPW)], src_v.at[pl.ds(0, EPW)],
            stsem[0])
        st2 = pltpu.make_async_copy(
            ei_hbm.at[0, pl.ds(wid * EPW, EPW)], dst_v.at[pl.ds(0, EPW)],
            stsem[1])
        st3 = pltpu.make_async_copy(
            adj_hbm.at[pl.ds(wid * EPW, EPW)], adj_v.at[pl.ds(0, EPW)],
            stsem[2])
        st1.start()
        st2.start()
        st3.start()
        if fused:
            st4 = pltpu.make_async_copy(
                sup_hbm.at[0, pl.ds(s * RPS, RPS)], pb0, stsem[3])
            st5 = pltpu.make_async_copy(
                sup_hbm.at[1, pl.ds(s * RPS, RPS)], pb1, stsem[4])
            st6 = pltpu.make_async_copy(w_hbm, wv, stsem[5])
        else:
            st4 = pltpu.make_async_copy(
                sup_hbm.at[pl.ds(s * RPS, RPS)],
                sup_sh.at[pl.ds(s * RPS, RPS)], stsem[3])
        st4.start()
        if fused:
            st5.start()
            st6.start()

        def zero_body(i, carry):
            drain_v[i, :] = jnp.zeros((W_COL,), jnp.float32)
            return carry
        lax.fori_loop(0, RPS, zero_body, 0)
        pltpu.sync_copy(drain_v, acc_sh.at[pl.ds(s * RPS, RPS)])
        st1.wait()
        st2.wait()
        st3.wait()
        st4.wait()
        if fused:
            st5.wait()
            st6.wait()
            wrows = [wv[k, :] for k in range(W_COL)]

            def mm_body(i, carry):
                r = i * 2
                ha = jnp.maximum(pb0[r, :] + pb1[r, :], 0.0)
                hb = jnp.maximum(pb0[r + 1, :] + pb1[r + 1, :], 0.0)
                acca = _splat(ha, 0) * wrows[0]
                accb = _splat(hb, 0) * wrows[0]
                for k in range(1, W_COL):
                    acca = acca + _splat(ha, k) * wrows[k]
                    accb = accb + _splat(hb, k) * wrows[k]
                drain_v[r, :] = acca
                drain_v[r + 1, :] = accb
                return carry
            lax.fori_loop(0, RPS // 2, mm_body, 0)
            pltpu.sync_copy(drain_v, sup_sh.at[pl.ds(s * RPS, RPS)])

        def start_gather(j, b):
            pltpu.make_async_copy(
                sup_sh.at[src_v.at[pl.ds(j * CHUNK, CHUNK)]],
                gbuf.at[b], gsem[b]).start()

        def wait_gather(j, b):
            pltpu.make_async_copy(
                sup_sh.at[src_v.at[pl.ds(j * CHUNK, CHUNK)]],
                gbuf.at[b], gsem[b]).wait()

        def start_scatter(j, b):
            pltpu.make_async_copy(
                sbuf.at[b], acc_sh.at[dst_v.at[pl.ds(j * CHUNK, CHUNK)]],
                ssem[b]).start(add=True)

        def wait_scatter(j, b):
            pltpu.make_async_copy(
                sbuf.at[b], acc_sh.at[dst_v.at[pl.ds(j * CHUNK, CHUNK)]],
                ssem[b]).wait()

        def scale(j, b):
            for g in range(CHUNK // W_COL):
                a = adj_v[pl.ds(j * CHUNK + g * W_COL, W_COL)]
                for l in range(W_COL):
                    e = g * W_COL + l
                    sbuf[b, e, :] = gbuf[b, e, :] * _splat(a, l)

        plsc.subcore_barrier()
        for b in range(NBUF):
            start_gather(b, b)

        for j in range(NBUF):
            b = j % NBUF
            wait_gather(j, b)
            scale(j, b)
            start_scatter(j, b)
            start_gather(j + NBUF, b)

        def steady(g, carry):
            for b in range(NBUF):
                j = NBUF + g * NBUF + b
                wait_gather(j, b)
                wait_scatter(j, b)
                scale(j, b)
                start_scatter(j, b)
                start_gather(j + NBUF, b)
            return carry
        lax.fori_loop(0, CPW // NBUF - 2, steady, 0)

        for j in range(CPW - NBUF, CPW):
            b = j % NBUF
            wait_gather(j, b)
            wait_scatter(j, b)
            scale(j, b)
            start_scatter(j, b)

        for j in range(CPW - NBUF, CPW):
            wait_scatter(j, j % NBUF)

        plsc.subcore_barrier()
        pltpu.sync_copy(acc_sh.at[pl.ds(s * RPS, RPS)], drain_v)
        pltpu.sync_copy(drain_v, out_hbm.at[c, pl.ds(s * RPS, RPS)])

    if fused:
        return spmm(edge_index, adj_vals, sup, w)
    return spmm(edge_index, adj_vals, sup)


def _mm_x_w1(x, w1):
    bm = 1000

    def body(x_ref, w_ref, o_ref):
        o_ref[:] = jnp.dot(x_ref[:], w_ref[:],
                           preferred_element_type=jnp.float32)

    return pl.pallas_call(
        body,
        grid=(N // bm,),
        in_specs=[
            pl.BlockSpec((bm, NFEAT), lambda i: (i, 0)),
            pl.BlockSpec((NFEAT, W_COL), lambda i: (0, 0)),
        ],
        out_specs=pl.BlockSpec((bm, W_COL), lambda i: (i, 0)),
        out_shape=jax.ShapeDtypeStruct((N_PAD, W_COL), jnp.float32),
    )(x, w1)


def _sum_log_softmax(p, nclass):
    bm = 1000

    def body(p_ref, o_ref):
        z = p_ref[0] + p_ref[1]
        col = lax.broadcasted_iota(jnp.int32, (bm, W_COL), 1)
        valid = col < nclass
        zm = jnp.where(valid, z, -jnp.inf)
        m = jnp.max(zm, axis=1, keepdims=True)
        ez = jnp.where(valid, jnp.exp(z - m), 0.0)
        ssum = jnp.sum(ez, axis=1, keepdims=True)
        o_ref[:] = (z - m - jnp.log(ssum))[:, :o_ref.shape[1]]

    return pl.pallas_call(
        body,
        grid=(N // bm,),
        in_specs=[pl.BlockSpec((NC, bm, W_COL), lambda i: (0, i, 0))],
        out_specs=pl.BlockSpec((bm, nclass), lambda i: (i, 0)),
        out_shape=jax.ShapeDtypeStruct((N, nclass), jnp.float32),
    )(p)


def kernel(x, edge_index, adj_vals, W1, W2, W3):
    nclass = W3.shape[1]
    w3p = jnp.pad(W3, ((0, 0), (0, W_COL - nclass)))

    sup = _mm_x_w1(x, W1)
    p = _spmm_sc(edge_index, adj_vals, sup)
    p = _spmm_sc(edge_index, adj_vals, p, W2)
    p = _spmm_sc(edge_index, adj_vals, p, w3p)
    return _sum_log_softmax(p, nclass)


# speedup vs baseline: 1.0694x; 1.0073x over previous
import functools

import jax
import jax.numpy as jnp
from jax import lax
from jax.experimental import pallas as pl
from jax.experimental.pallas import tpu as pltpu
from jax.experimental.pallas import tpu_sc as plsc

N = 10000
N_PAD = 10240
E = 160000
NFEAT = 500
W_COL = 16

NC = 2
NS = 16
NW = NC * NS
CHUNK = 128
EPW = E // NW
CPW = (EPW + CHUNK - 1) // CHUNK
EBUF = CPW * CHUNK
RPS = N_PAD // NS
NBUF = 4


def _splat(vec, lane):
    return lax.gather(
        vec,
        jnp.full((W_COL, 1), lane, jnp.int32),
        lax.GatherDimensionNumbers(
            offset_dims=(),
            collapsed_slice_dims=(0,),
            start_index_map=(0,),
        ),
        slice_sizes=(1,),
        mode=lax.GatherScatterMode.PROMISE_IN_BOUNDS,
    )


def _spmm_sc(edge_index, adj_vals, sup, w=None):
    fused = w is not None
    mesh = plsc.VectorSubcoreMesh(
        core_axis_name="c", subcore_axis_name="s", num_cores=NC, num_subcores=NS
    )

    scratch = [
        pltpu.VMEM((EBUF,), jnp.int32),
        pltpu.VMEM((EBUF,), jnp.int32),
        pltpu.VMEM((EBUF,), jnp.float32),
        pltpu.VMEM((NBUF, CHUNK, W_COL), jnp.float32),
        pltpu.VMEM((NBUF, CHUNK, W_COL), jnp.float32),
        pltpu.VMEM((RPS, W_COL), jnp.float32),
        pltpu.VMEM_SHARED((N_PAD, W_COL), jnp.float32),
        pltpu.VMEM_SHARED((N_PAD, W_COL), jnp.float32),
        [pltpu.SemaphoreType.DMA] * NBUF,
        [pltpu.SemaphoreType.DMA] * NBUF,
        [pltpu.SemaphoreType.DMA] * 6,
    ]
    if fused:
        scratch += [
            pltpu.VMEM((RPS, W_COL), jnp.float32),
            pltpu.VMEM((RPS, W_COL), jnp.float32),
            pltpu.VMEM((W_COL, W_COL), jnp.float32),
        ]

    @functools.partial(
        pl.kernel,
        out_type=jax.ShapeDtypeStruct((NC, N_PAD, W_COL), jnp.float32),
        mesh=mesh,
        compiler_params=pltpu.CompilerParams(use_tc_tiling_on_sc=False),
        scratch_types=scratch,
    )
    def spmm(ei_hbm, adj_hbm, sup_hbm, *rest):
        if fused:
            (w_hbm, out_hbm,
             src_v, dst_v, adj_v, gbuf, sbuf, drain_v, acc_sh, sup_sh,
             gsem, ssem, stsem, pb0, pb1, wv) = rest
        else:
            (out_hbm,
             src_v, dst_v, adj_v, gbuf, sbuf, drain_v, acc_sh, sup_sh,
             gsem, ssem, stsem) = rest
        c = lax.axis_index("c")
        s = lax.axis_index("s")
        wid = s * NC + c

        tail0 = (EPW // W_COL) * W_COL
        for t in range(tail0, EBUF, W_COL):
            src_v[pl.ds(t, W_COL)] = jnp.zeros((W_COL,), jnp.int32)
            dst_v[pl.ds(t, W_COL)] = jnp.zeros((W_COL,), jnp.int32)
            adj_v[pl.ds(t, W_COL)] = jnp.zeros((W_COL,), jnp.float32)
        st1 = pltpu.make_async_copy(
            ei_hbm.at[1, pl.ds(wid * EPW, EPW)], src_v.at[pl.ds(0, EPW)],
            stsem[0])
        st2 = pltpu.make_async_copy(
            ei_hbm.at[0, pl.ds(wid * EPW, EPW)], dst_v.at[pl.ds(0, EPW)],
            stsem[1])
        st3 = pltpu.make_async_copy(
            adj_hbm.at[pl.ds(wid * EPW, EPW)], adj_v.at[pl.ds(0, EPW)],
            stsem[2])
        st1.start()
        st2.start()
        st3.start()
        if fused:
            st4 = pltpu.make_async_copy(
                sup_hbm.at[0, pl.ds(s * RPS, RPS)], pb0, stsem[3])
            st5 = pltpu.make_async_copy(
                sup_hbm.at[1, pl.ds(s * RPS, RPS)], pb1, stsem[4])
            st6 = pltpu.make_async_copy(w_hbm, wv, stsem[5])
        else:
            st4 = pltpu.make_async_copy(
                sup_hbm.at[pl.ds(s * RPS, RPS)],
                sup_sh.at[pl.ds(s * RPS, RPS)], stsem[3])
        st4.start()
        if fused:
            st5.start()
            st6.start()

        def zero_body(i, carry):
            drain_v[i, :] = jnp.zeros((W_COL,), jnp.float32)
            return carry
        lax.fori_loop(0, RPS, zero_body, 0)
        pltpu.sync_copy(drain_v, acc_sh.at[pl.ds(s * RPS, RPS)])
        st1.wait()
        st2.wait()
        st3.wait()
        st4.wait()
        if fused:
            st5.wait()
            st6.wait()
            wrows = [wv[k, :] for k in range(W_COL)]

            def mm_body(r, carry):
                h = jnp.maximum(pb0[r, :] + pb1[r, :], 0.0)
                acc = _splat(h, 0) * wrows[0]
                for k in range(1, W_COL):
                    acc = acc + _splat(h, k) * wrows[k]
                drain_v[r, :] = acc
                return carry
            lax.fori_loop(0, RPS, mm_body, 0)
            pltpu.sync_copy(drain_v, sup_sh.at[pl.ds(s * RPS, RPS)])

        def start_gather(j, b):
            pltpu.make_async_copy(
                sup_sh.at[src_v.at[pl.ds(j * CHUNK, CHUNK)]],
                gbuf.at[b], gsem[b]).start()

        def wait_gather(j, b):
            pltpu.make_async_copy(
                sup_sh.at[src_v.at[pl.ds(j * CHUNK, CHUNK)]],
                gbuf.at[b], gsem[b]).wait()

        def start_scatter(j, b):
            pltpu.make_async_copy(
                sbuf.at[b], acc_sh.at[dst_v.at[pl.ds(j * CHUNK, CHUNK)]],
                ssem[b]).start(add=True)

        def wait_scatter(j, b):
            pltpu.make_async_copy(
                sbuf.at[b], acc_sh.at[dst_v.at[pl.ds(j * CHUNK, CHUNK)]],
                ssem[b]).wait()

        def scale(j, b):
            for g in range(CHUNK // W_COL):
                a = adj_v[pl.ds(j * CHUNK + g * W_COL, W_COL)]
                for l in range(W_COL):
                    e = g * W_COL + l
                    sbuf[b, e, :] = gbuf[b, e, :] * _splat(a, l)

        plsc.subcore_barrier()
        for b in range(NBUF):
            start_gather(b, b)

        for j in range(NBUF):
            b = j % NBUF
            wait_gather(j, b)
            scale(j, b)
            start_scatter(j, b)
            start_gather(j + NBUF, b)

        def steady(g, carry):
            for b in range(NBUF):
                j = NBUF + g * NBUF + b
                wait_gather(j, b)
                wait_scatter(j, b)
                scale(j, b)
                start_scatter(j, b)
                start_gather(j + NBUF, b)
            return carry
        lax.fori_loop(0, CPW // NBUF - 2, steady, 0)

        for j in range(CPW - NBUF, CPW):
            b = j % NBUF
            wait_gather(j, b)
            wait_scatter(j, b)
            scale(j, b)
            start_scatter(j, b)

        for j in range(CPW - NBUF, CPW):
            wait_scatter(j, j % NBUF)

        plsc.subcore_barrier()
        pltpu.sync_copy(acc_sh.at[pl.ds(s * RPS, RPS)], drain_v)
        pltpu.sync_copy(drain_v, out_hbm.at[c, pl.ds(s * RPS, RPS)])

    if fused:
        return spmm(edge_index, adj_vals, sup, w)
    return spmm(edge_index, adj_vals, sup)


def _mm_x_w1(x, w1):
    bm = 1000

    def body(x_ref, w_ref, o_ref):
        o_ref[:] = jnp.dot(x_ref[:], w_ref[:],
                           preferred_element_type=jnp.float32)

    return pl.pallas_call(
        body,
        grid=(N // bm,),
        in_specs=[
            pl.BlockSpec((bm, NFEAT), lambda i: (i, 0)),
            pl.BlockSpec((NFEAT, W_COL), lambda i: (0, 0)),
        ],
        out_specs=pl.BlockSpec((bm, W_COL), lambda i: (i, 0)),
        out_shape=jax.ShapeDtypeStruct((N_PAD, W_COL), jnp.float32),
    )(x, w1)


def _sum_log_softmax(p, nclass):
    bm = 1000

    def body(p_ref, o_ref):
        z = p_ref[0] + p_ref[1]
        col = lax.broadcasted_iota(jnp.int32, (bm, W_COL), 1)
        valid = col < nclass
        zm = jnp.where(valid, z, -jnp.inf)
        m = jnp.max(zm, axis=1, keepdims=True)
        ez = jnp.where(valid, jnp.exp(z - m), 0.0)
        ssum = jnp.sum(ez, axis=1, keepdims=True)
        o_ref[:] = (z - m - jnp.log(ssum))[:, :o_ref.shape[1]]

    return pl.pallas_call(
        body,
        grid=(N // bm,),
        in_specs=[pl.BlockSpec((NC, bm, W_COL), lambda i: (0, i, 0))],
        out_specs=pl.BlockSpec((bm, nclass), lambda i: (i, 0)),
        out_shape=jax.ShapeDtypeStruct((N, nclass), jnp.float32),
    )(p)


def kernel(x, edge_index, adj_vals, W1, W2, W3):
    nclass = W3.shape[1]
    w3p = jnp.pad(W3, ((0, 0), (0, W_COL - nclass)))

    sup = _mm_x_w1(x, W1)
    p = _spmm_sc(edge_index, adj_vals, sup)
    p = _spmm_sc(edge_index, adj_vals, p, W2)
    p = _spmm_sc(edge_index, adj_vals, p, w3p)
    return _sum_log_softmax(p, nclass)
